# baseline (speedup 1.0000x reference)
import functools

import jax
import jax.numpy as jnp
from jax import lax
from jax.experimental import pallas as pl
from jax.experimental.pallas import tpu as pltpu
from jax.experimental.pallas import tpu_sc as plsc

N_ROWS = 320000
D = 128
N_SEG = 512
N_WORKERS = 32
ROWS_PER_W = N_ROWS // N_WORKERS
CHUNK = 40
NCH = ROWS_PER_W // CHUNK
GROUPS = 62
N_STREAM = 125
IDX_STAGE = 128
SEG_PER_SUB = N_SEG // 16
NSBUF = 4
NTBUF = 2
MERGE_B = N_SEG // 128


def _sc_body(h_hbm, idx_hbm, iota_hbm, out_hbm, *sc):
    n = NSBUF + NTBUF
    bufs = sc[:n]
    idx_v, idxt0, idxt1, idx_id, acc_l, acc_sh = sc[n:n + 6]
    lsems = sc[n + 6:2 * n + 6]
    ssems = sc[2 * n + 6:2 * n + 6 + NSBUF]
    tsems = sc[2 * n + 6 + NSBUF:]
    idxts = (idxt0, idxt1)
    core = lax.axis_index("c")
    sub = lax.axis_index("s")
    wid = core * 16 + sub
    row_base = wid * ROWS_PER_W

    def zacc(r, _):
        for k in range(D // 16):
            acc_l[r, pl.ds(k * 16, 16)] = jnp.zeros((16,), jnp.float32)
        return 0
    lax.fori_loop(0, N_SEG, zacc, 0)
    pltpu.sync_copy(acc_l.at[pl.ds(0, SEG_PER_SUB)],
                    acc_sh.at[pl.ds(sub * SEG_PER_SUB, SEG_PER_SUB)])

    pltpu.sync_copy(idx_hbm.at[wid, pl.ds(0, IDX_STAGE)], idx_v)
    pltpu.sync_copy(iota_hbm, idx_id)

    plsc.subcore_barrier()

    def load_start(c, b):
        pltpu.async_copy(
            h_hbm.at[pl.ds(row_base + c * CHUNK, CHUNK)], bufs[b], lsems[b])

    def load_wait(c, b):
        pltpu.make_async_copy(
            h_hbm.at[pl.ds(row_base + c * CHUNK, CHUNK)], bufs[b],
            lsems[b]).wait()

    def scat_start(s, b):
        pltpu.async_copy(bufs[b], acc_sh.at[idx_v.at[s]], ssems[b], add=True)

    def scat_wait(s, b):
        pltpu.make_async_copy(
            bufs[b], acc_sh.at[idx_v.at[s]], ssems[b]).wait()

    def tidx_start(t, q):
        pltpu.async_copy(idx_hbm.at[wid, N_STREAM + t], idxts[q], tsems[q])

    def tidx_wait(t, q):
        pltpu.make_async_copy(idx_hbm.at[wid, N_STREAM + t], idxts[q],
                              tsems[q]).wait()

    def fold_chunk(t, q):
        buf = bufs[NSBUF + q]
        idxt = idxts[q]
        w0 = idxt[pl.ds(0, 16)]
        w1 = idxt[pl.ds(16, 16)]
        w2 = idxt[pl.ds(CHUNK - 16, 16)]
        seg0 = w0[0]
        uniform = w2[15] == seg0

        @pl.when(uniform)
        def _():
            for k in range(D // 16):
                dsk = pl.ds(k * 16, 16)
                xs = [buf[r, dsk] for r in range(CHUNK)]
                while len(xs) > 1:
                    xs = [xs[i] + xs[i + 1] for i in range(0, len(xs) - 1, 2)]                         + ([xs[-1]] if len(xs) % 2 else [])
                plsc.addupdate(acc_l.at[seg0, dsk], xs[0])

        @pl.when(jnp.logical_not(uniform))
        def _():
            def slow16(seg_vec, r0, n0=0):
                for u in range(n0, 16):
                    r = r0 + u
                    seg = seg_vec[u]
                    for k in range(D // 16):
                        plsc.addupdate(
                            acc_l.at[seg, pl.ds(k * 16, 16)],
                            buf[r, pl.ds(k * 16, 16)])
            slow16(w0, 0)
            slow16(w1, 16)
            slow16(w2, CHUNK - 16, n0=8)

    def tec_chunk(t):
        return N_STREAM + t

    load_start(0, 0)
    load_start(1, 1)
    load_start(tec_chunk(0), NSBUF)
    tidx_start(0, 0)
    load_start(tec_chunk(1), NSBUF + 1)
    tidx_start(1, 1)

    def pair_body(i, _):
        for p in range(2):
            g = 2 * i + p
            sb0, sb1 = (0, 1) if p == 0 else (2, 3)
            pb0, pb1 = (2, 3) if p == 0 else (0, 1)
            if p == 0:
                @pl.when(g >= 1)
                def _(g=g, pb0=pb0, pb1=pb1):
                    scat_wait(2 * g - 2, pb0)
                    scat_wait(2 * g - 1, pb1)
            else:
                scat_wait(2 * g - 2, pb0)
                scat_wait(2 * g - 1, pb1)
            load_start(2 * g + 2, pb0)

            @pl.when(g < GROUPS - 1)
            def _(g=g, pb1=pb1):
                load_start(2 * g + 3, pb1)
            load_wait(2 * g, sb0)
            scat_start(2 * g, sb0)
            load_wait(2 * g + 1, sb1)
            scat_start(2 * g + 1, sb1)
            load_wait(tec_chunk(2 * g), NSBUF)
            tidx_wait(2 * g, 0)
            fold_chunk(2 * g, 0)
            load_start(tec_chunk(2 * g + 2), NSBUF)
            tidx_start(2 * g + 2, 0)
            load_wait(tec_chunk(2 * g + 1), NSBUF + 1)
            tidx_wait(2 * g + 1, 1)
            fold_chunk(2 * g + 1, 1)

            @pl.when(g < GROUPS - 1)
            def _(g=g):
                load_start(tec_chunk(2 * g + 3), NSBUF + 1)
                tidx_start(2 * g + 3, 1)
        return 0

    lax.fori_loop(0, GROUPS // 2, pair_body, 0)

    scat_wait(2 * GROUPS - 2, (2 * GROUPS - 2) % NSBUF)
    scat_wait(2 * GROUPS - 1, (2 * GROUPS - 1) % NSBUF)
    load_wait(2 * GROUPS, (2 * GROUPS) % NSBUF)
    scat_start(2 * GROUPS, (2 * GROUPS) % NSBUF)
    load_wait(tec_chunk(2 * GROUPS), NSBUF)
    tidx_wait(2 * GROUPS, 0)
    fold_chunk(2 * GROUPS, 0)
    scat_wait(2 * GROUPS, (2 * GROUPS) % NSBUF)

    plsc.subcore_barrier()

    for k in range(MERGE_B):
        pltpu.sync_copy(
            acc_l.at[pl.ds(k * 128, 128)], acc_sh.at[idx_id.at[k]], add=True)

    plsc.subcore_barrier()

    pltpu.sync_copy(
        acc_sh.at[pl.ds(sub * SEG_PER_SUB, SEG_PER_SUB)],
        out_hbm.at[core, pl.ds(sub * SEG_PER_SUB, SEG_PER_SUB)])


_sc_segsum = functools.partial(
    pl.kernel,
    out_type=jax.ShapeDtypeStruct((2, N_SEG, D), jnp.float32),
    mesh=plsc.VectorSubcoreMesh(core_axis_name="c", subcore_axis_name="s"),
    scratch_types=(
        [pltpu.VMEM((CHUNK, D), jnp.float32) for _ in range(NSBUF + NTBUF)]
        + [
            pltpu.VMEM((IDX_STAGE, CHUNK), jnp.int32),
            pltpu.VMEM((CHUNK,), jnp.int32),
            pltpu.VMEM((CHUNK,), jnp.int32),
            pltpu.VMEM((MERGE_B, 128), jnp.int32),
            pltpu.VMEM((N_SEG, D), jnp.float32),
            pltpu.VMEM_SHARED((N_SEG, D), jnp.float32),
        ]
        + [pltpu.SemaphoreType.DMA for _ in range(NSBUF + NTBUF)]
        + [pltpu.SemaphoreType.DMA for _ in range(NSBUF)]
        + [pltpu.SemaphoreType.DMA for _ in range(NTBUF)]
    ),
)(_sc_body)


def _merge_body(p_ref, o_ref):
    o_ref[...] = p_ref[0] + p_ref[1]


def _merge(partials):
    return pl.pallas_call(
        _merge_body,
        out_shape=jax.ShapeDtypeStruct((N_SEG, D), jnp.float32),
    )(partials)


@jax.jit
def kernel(h, index):
    idx = index.astype(jnp.int32).reshape(N_WORKERS, NCH, CHUNK)
    iota = jnp.arange(N_SEG, dtype=jnp.int32).reshape(MERGE_B, 128)
    partials = _sc_segsum(h, idx, iota)
    return _merge(partials)


# speedup vs baseline: 1.0892x; 1.0892x over previous
import functools

import jax
import jax.numpy as jnp
from jax import lax
from jax.experimental import pallas as pl
from jax.experimental.pallas import tpu as pltpu
from jax.experimental.pallas import tpu_sc as plsc

N_ROWS = 320000
D = 128
N_SEG = 512
N_WORKERS = 32
ROWS_PER_W = N_ROWS // N_WORKERS
CHUNK = 40
NCH = ROWS_PER_W // CHUNK
GROUPS = 50
N_STREAM = 150
N_TEC = NCH - N_STREAM
IDX_STAGE = 152
SEG_PER_SUB = N_SEG // 16
NSBUF = 6
NTBUF = 2
MERGE_B = N_SEG // 128


def _sc_body(h_hbm, idx_hbm, iota_hbm, out_hbm, *sc):
    n = NSBUF + NTBUF
    bufs = sc[:n]
    idx_v, idxt0, idxt1, idx_id, acc_l, acc_sh = sc[n:n + 6]
    lsems = sc[n + 6:2 * n + 6]
    ssems = sc[2 * n + 6:2 * n + 6 + NSBUF]
    tsems = sc[2 * n + 6 + NSBUF:]
    idxts = (idxt0, idxt1)
    core = lax.axis_index("c")
    sub = lax.axis_index("s")
    wid = core * 16 + sub
    row_base = wid * ROWS_PER_W

    def zacc(r, _):
        for k in range(D // 16):
            acc_l[r, pl.ds(k * 16, 16)] = jnp.zeros((16,), jnp.float32)
        return 0
    lax.fori_loop(0, N_SEG, zacc, 0)
    pltpu.sync_copy(acc_l.at[pl.ds(0, SEG_PER_SUB)],
                    acc_sh.at[pl.ds(sub * SEG_PER_SUB, SEG_PER_SUB)])

    pltpu.sync_copy(idx_hbm.at[wid, pl.ds(0, IDX_STAGE)], idx_v)
    pltpu.sync_copy(iota_hbm, idx_id)

    plsc.subcore_barrier()

    def load_start(c, b):
        pltpu.async_copy(
            h_hbm.at[pl.ds(row_base + c * CHUNK, CHUNK)], bufs[b], lsems[b])

    def load_wait(c, b):
        pltpu.make_async_copy(
            h_hbm.at[pl.ds(row_base + c * CHUNK, CHUNK)], bufs[b],
            lsems[b]).wait()

    def scat_start(s, b):
        pltpu.async_copy(bufs[b], acc_sh.at[idx_v.at[s]], ssems[b], add=True)

    def scat_wait(s, b):
        pltpu.make_async_copy(
            bufs[b], acc_sh.at[idx_v.at[s]], ssems[b]).wait()

    def tidx_start(t, q):
        pltpu.async_copy(idx_hbm.at[wid, N_STREAM + t], idxts[q], tsems[q])

    def tidx_wait(t, q):
        pltpu.make_async_copy(idx_hbm.at[wid, N_STREAM + t], idxts[q],
                              tsems[q]).wait()

    def fold_chunk(t, q):
        buf = bufs[NSBUF + q]
        idxt = idxts[q]
        w0 = idxt[pl.ds(0, 16)]
        w1 = idxt[pl.ds(16, 16)]
        w2 = idxt[pl.ds(CHUNK - 16, 16)]
        seg0 = w0[0]
        uniform = w2[15] == seg0

        @pl.when(uniform)
        def _():
            for k in range(D // 16):
                dsk = pl.ds(k * 16, 16)
                xs = [buf[r, dsk] for r in range(CHUNK)]
                while len(xs) > 1:
                    xs = [xs[i] + xs[i + 1] for i in range(0, len(xs) - 1, 2)]                         + ([xs[-1]] if len(xs) % 2 else [])
                plsc.addupdate(acc_l.at[seg0, dsk], xs[0])

        @pl.when(jnp.logical_not(uniform))
        def _():
            def slow16(seg_vec, r0, n0=0):
                for u in range(n0, 16):
                    r = r0 + u
                    seg = seg_vec[u]
                    for k in range(D // 16):
                        plsc.addupdate(
                            acc_l.at[seg, pl.ds(k * 16, 16)],
                            buf[r, pl.ds(k * 16, 16)])
            slow16(w0, 0)
            slow16(w1, 16)
            slow16(w2, CHUNK - 16, n0=8)

    def tec_chunk(t):
        return N_STREAM + t

    load_start(0, 0)
    load_start(1, 1)
    load_start(2, 2)
    load_start(tec_chunk(0), NSBUF)
    tidx_start(0, 0)
    load_start(tec_chunk(1), NSBUF + 1)
    tidx_start(1, 1)

    def pair_body(i, _):
        for p in range(2):
            g = 2 * i + p
            sbs = (0, 1, 2) if p == 0 else (3, 4, 5)
            pbs = (3, 4, 5) if p == 0 else (0, 1, 2)

            def prefetch_stream(g=g, pbs=pbs):
                for j in range(3):
                    @pl.when(g >= 1)
                    def _(g=g, j=j, pbs=pbs):
                        scat_wait(3 * g - 3 + j, pbs[j])
                    load_start(3 * g + 3 + j, pbs[j])
            if p == 0:
                prefetch_stream()
            else:
                pl.when(g < GROUPS - 1)(prefetch_stream)
            for j in range(3):
                load_wait(3 * g + j, sbs[j])
                scat_start(3 * g + j, sbs[j])
            load_wait(tec_chunk(2 * g), NSBUF)
            tidx_wait(2 * g, 0)
            fold_chunk(2 * g, 0)

            @pl.when(g < GROUPS - 1)
            def _(g=g):
                load_start(tec_chunk(2 * g + 2), NSBUF)
                tidx_start(2 * g + 2, 0)
            load_wait(tec_chunk(2 * g + 1), NSBUF + 1)
            tidx_wait(2 * g + 1, 1)
            fold_chunk(2 * g + 1, 1)

            @pl.when(g < GROUPS - 1)
            def _(g=g):
                load_start(tec_chunk(2 * g + 3), NSBUF + 1)
                tidx_start(2 * g + 3, 1)
        return 0

    lax.fori_loop(0, GROUPS // 2, pair_body, 0)

    for j in range(6):
        scat_wait(3 * GROUPS - 6 + j, j)

    plsc.subcore_barrier()

    for k in range(MERGE_B):
        pltpu.sync_copy(
            acc_l.at[pl.ds(k * 128, 128)], acc_sh.at[idx_id.at[k]], add=True)

    plsc.subcore_barrier()

    pltpu.sync_copy(
        acc_sh.at[pl.ds(sub * SEG_PER_SUB, SEG_PER_SUB)],
        out_hbm.at[core, pl.ds(sub * SEG_PER_SUB, SEG_PER_SUB)])


_sc_segsum = functools.partial(
    pl.kernel,
    out_type=jax.ShapeDtypeStruct((2, N_SEG, D), jnp.float32),
    mesh=plsc.VectorSubcoreMesh(core_axis_name="c", subcore_axis_name="s"),
    scratch_types=(
        [pltpu.VMEM((CHUNK, D), jnp.float32) for _ in range(NSBUF + NTBUF)]
        + [
            pltpu.VMEM((IDX_STAGE, CHUNK), jnp.int32),
            pltpu.VMEM((CHUNK,), jnp.int32),
            pltpu.VMEM((CHUNK,), jnp.int32),
            pltpu.VMEM((MERGE_B, 128), jnp.int32),
            pltpu.VMEM((N_SEG, D), jnp.float32),
            pltpu.VMEM_SHARED((N_SEG, D), jnp.float32),
        ]
        + [pltpu.SemaphoreType.DMA for _ in range(NSBUF + NTBUF)]
        + [pltpu.SemaphoreType.DMA for _ in range(NSBUF)]
        + [pltpu.SemaphoreType.DMA for _ in range(NTBUF)]
    ),
)(_sc_body)


def _merge_body(p_ref, o_ref):
    o_ref[...] = p_ref[0] + p_ref[1]


def _merge(partials):
    return pl.pallas_call(
        _merge_body,
        out_shape=jax.ShapeDtypeStruct((N_SEG, D), jnp.float32),
    )(partials)


@jax.jit
def kernel(h, index):
    idx = index.astype(jnp.int32).reshape(N_WORKERS, NCH, CHUNK)
    iota = jnp.arange(N_SEG, dtype=jnp.int32).reshape(MERGE_B, 128)
    partials = _sc_segsum(h, idx, iota)
    return _merge(partials)


# speedup vs baseline: 1.2770x; 1.1725x over previous
import functools

import jax
import jax.numpy as jnp
from jax import lax
from jax.experimental import pallas as pl
from jax.experimental.pallas import tpu as pltpu
from jax.experimental.pallas import tpu_sc as plsc

N_ROWS = 320000
D = 128
N_SEG = 512
N_WORKERS = 32
ROWS_PER_W = N_ROWS // N_WORKERS
CHUNK = 40
NCH = ROWS_PER_W // CHUNK
GROUPS = 83
N_STREAM = 2 * GROUPS + 1
SEG_PER_SUB = N_SEG // 16
NSBUF = 4
NTBUF = 2
MERGE_B = N_SEG // 128


def _sc_body(h_hbm, idx_hbm, iota_hbm, out_hbm, *sc):
    n = NSBUF + NTBUF
    bufs = sc[:n]
    idx_v, idxt0, idxt1, idx_id, acc_l, acc_sh = sc[n:n + 6]
    lsems = sc[n + 6:2 * n + 6]
    ssems = sc[2 * n + 6:2 * n + 6 + NSBUF]
    tsems = sc[2 * n + 6 + NSBUF:]
    idxts = (idxt0, idxt1)
    core = lax.axis_index("c")
    sub = lax.axis_index("s")
    wid = core * 16 + sub
    row_base = wid * ROWS_PER_W

    def zacc(r, _):
        for k in range(D // 16):
            acc_l[r, pl.ds(k * 16, 16)] = jnp.zeros((16,), jnp.float32)
        return 0
    lax.fori_loop(0, N_SEG, zacc, 0)
    pltpu.sync_copy(acc_l.at[pl.ds(0, SEG_PER_SUB)],
                    acc_sh.at[pl.ds(sub * SEG_PER_SUB, SEG_PER_SUB)])

    pltpu.sync_copy(idx_hbm.at[wid, pl.ds(0, N_STREAM + 1)], idx_v)
    pltpu.sync_copy(iota_hbm, idx_id)

    plsc.subcore_barrier()

    def load_start(c, b):
        pltpu.async_copy(
            h_hbm.at[pl.ds(row_base + c * CHUNK, CHUNK)], bufs[b], lsems[b])

    def load_wait(c, b):
        pltpu.make_async_copy(
            h_hbm.at[pl.ds(row_base + c * CHUNK, CHUNK)], bufs[b],
            lsems[b]).wait()

    def scat_start(s, b):
        pltpu.async_copy(bufs[b], acc_sh.at[idx_v.at[s]], ssems[b], add=True)

    def scat_wait(s, b):
        pltpu.make_async_copy(
            bufs[b], acc_sh.at[idx_v.at[s]], ssems[b]).wait()

    def tidx_start(t, q):
        pltpu.async_copy(idx_hbm.at[wid, N_STREAM + t], idxts[q], tsems[q])

    def tidx_wait(t, q):
        pltpu.make_async_copy(idx_hbm.at[wid, N_STREAM + t], idxts[q],
                              tsems[q]).wait()

    def fold_chunk(t, q):
        buf = bufs[NSBUF + q]
        idxt = idxts[q]

        def fold16(seg_vec, r0, n0=0):
            for u in range(n0, 16):
                r = r0 + u
                seg = seg_vec[u]
                for k in range(D // 16):
                    plsc.addupdate(
                        acc_l.at[seg, pl.ds(k * 16, 16)],
                        buf[r, pl.ds(k * 16, 16)])

        def rows16(j, _):
            fold16(idxt[pl.ds(j * 16, 16)], j * 16)
            return 0
        lax.fori_loop(0, 2, rows16, 0)
        fold16(idxt[pl.ds(CHUNK - 16, 16)], CHUNK - 16, n0=8)

    def tec_chunk(t):
        return N_STREAM + t

    load_start(0, 0)
    load_start(1, 1)
    load_start(tec_chunk(0), NSBUF)
    tidx_start(0, 0)

    def pair_body(i, _):
        for p in range(2):
            g = 2 * i + p
            sb0, sb1 = (0, 1) if p == 0 else (2, 3)
            pb0, pb1 = (2, 3) if p == 0 else (0, 1)
            if p == 0:
                @pl.when(g >= 1)
                def _(g=g, pb0=pb0, pb1=pb1):
                    scat_wait(2 * g - 2, pb0)
                    scat_wait(2 * g - 1, pb1)
            else:
                scat_wait(2 * g - 2, pb0)
                scat_wait(2 * g - 1, pb1)
            load_start(2 * g + 2, pb0)
            load_start(2 * g + 3, pb1)
            load_start(tec_chunk(g + 1), NSBUF + 1 - p)
            tidx_start(g + 1, 1 - p)
            load_wait(2 * g, sb0)
            scat_start(2 * g, sb0)
            load_wait(2 * g + 1, sb1)
            scat_start(2 * g + 1, sb1)
            load_wait(tec_chunk(g), NSBUF + p)
            tidx_wait(g, p)
            fold_chunk(g, p)
        return 0

    lax.fori_loop(0, (GROUPS - 1) // 2, pair_body, 0)

    gl = GROUPS - 1
    scat_wait(2 * gl - 2, 2)
    scat_wait(2 * gl - 1, 3)
    load_start(2 * gl + 2, 2)
    load_wait(2 * gl, 0)
    scat_start(2 * gl, 0)
    load_wait(2 * gl + 1, 1)
    scat_start(2 * gl + 1, 1)
    load_wait(tec_chunk(gl), NSBUF)
    tidx_wait(gl, 0)
    fold_chunk(gl, 0)
    scat_wait(2 * gl, 0)
    scat_wait(2 * gl + 1, 1)
    load_wait(2 * gl + 2, 2)
    scat_start(2 * gl + 2, 2)
    scat_wait(2 * gl + 2, 2)

    plsc.subcore_barrier()

    for k in range(MERGE_B):
        pltpu.sync_copy(
            acc_l.at[pl.ds(k * 128, 128)], acc_sh.at[idx_id.at[k]], add=True)

    plsc.subcore_barrier()

    pltpu.sync_copy(
        acc_sh.at[pl.ds(sub * SEG_PER_SUB, SEG_PER_SUB)],
        out_hbm.at[core, pl.ds(sub * SEG_PER_SUB, SEG_PER_SUB)])


_sc_segsum = functools.partial(
    pl.kernel,
    out_type=jax.ShapeDtypeStruct((2, N_SEG, D), jnp.float32),
    mesh=plsc.VectorSubcoreMesh(core_axis_name="c", subcore_axis_name="s"),
    scratch_types=(
        [pltpu.VMEM((CHUNK, D), jnp.float32) for _ in range(NSBUF + NTBUF)]
        + [
            pltpu.VMEM((N_STREAM + 1, CHUNK), jnp.int32),
            pltpu.VMEM((CHUNK,), jnp.int32),
            pltpu.VMEM((CHUNK,), jnp.int32),
            pltpu.VMEM((MERGE_B, 128), jnp.int32),
            pltpu.VMEM((N_SEG, D), jnp.float32),
            pltpu.VMEM_SHARED((N_SEG, D), jnp.float32),
        ]
        + [pltpu.SemaphoreType.DMA for _ in range(NSBUF + NTBUF)]
        + [pltpu.SemaphoreType.DMA for _ in range(NSBUF)]
        + [pltpu.SemaphoreType.DMA for _ in range(NTBUF)]
    ),
)(_sc_body)


def _merge_body(p_ref, o_ref):
    o_ref[...] = p_ref[0] + p_ref[1]


def _merge(partials):
    return pl.pallas_call(
        _merge_body,
        out_shape=jax.ShapeDtypeStruct((N_SEG, D), jnp.float32),
    )(partials)


@jax.jit
def kernel(h, index):
    idx = index.astype(jnp.int32).reshape(N_WORKERS, NCH, CHUNK)
    iota = jnp.arange(N_SEG, dtype=jnp.int32).reshape(MERGE_B, 128)
    partials = _sc_segsum(h, idx, iota)
    return _merge(partials)
